# Initial kernel scaffold; baseline (speedup 1.0000x reference)
import functools

import jax
import jax.numpy as jnp
from jax import lax
from jax.experimental import pallas as pl
from jax.experimental.pallas import tpu as pltpu

B, D, H, E, L, K = 512, 768, 768, 8, 2, 2
F = 4 * D

_DN_T = (((1,), (1,)), ((), ()))


def _mm_t(a, w):
    return lax.dot_general(a, w, _DN_T, preferred_element_type=jnp.float32)


def _dense_body(x_ref, Wp_ref, bp_ref, Wih_ref, Whh_ref, bih_ref, bhh_ref,
                Wr_ref, br_ref, W1_ref, b1_ref, W2_ref, b2_ref,
                out_ref, xcur, h, wcomb, eo):
    l = pl.program_id(0)
    e = pl.program_id(1)

    @pl.when(jnp.logical_and(l == 0, e == 0))
    def _init():
        xcur[...] = x_ref[...]
        h[...] = jnp.zeros_like(h)

    @pl.when(e == 0)
    def _router():
        xf = xcur[...]
        xp = _mm_t(xf, Wp_ref[0]) + bp_ref[0]
        gi = _mm_t(xp, Wih_ref[...]) + bih_ref[0]
        gh = _mm_t(h[...], Whh_ref[...]) + bhh_ref[0]
        i_r, i_z, i_n = gi[:, :H], gi[:, H:2 * H], gi[:, 2 * H:]
        h_r, h_z, h_n = gh[:, :H], gh[:, H:2 * H], gh[:, 2 * H:]
        r = jax.nn.sigmoid(i_r + h_r)
        z = jax.nn.sigmoid(i_z + h_z)
        n = jnp.tanh(i_n + r * h_n)
        hn = (1.0 - z) * n + z * h[...]
        h[...] = hn

        logits = _mm_t(hn, Wr_ref[0]) + br_ref[0]
        m = jnp.max(logits, axis=-1, keepdims=True)
        p = jnp.exp(logits - m)
        p = p / jnp.sum(p, axis=-1, keepdims=True)
        ii = lax.broadcasted_iota(jnp.int32, (B, E), 1)
        m1 = jnp.max(p, axis=-1, keepdims=True)
        i1 = jnp.min(jnp.where(p >= m1, ii, E), axis=-1, keepdims=True)
        p2 = jnp.where(ii == i1, -1e30, p)
        m2 = jnp.max(p2, axis=-1, keepdims=True)
        i2 = jnp.min(jnp.where(p2 >= m2, ii, E), axis=-1, keepdims=True)
        s = m1 + m2
        wcomb[...] = (jnp.where(ii == i1, m1 / s, 0.0)
                      + jnp.where(ii == i2, m2 / s, 0.0))
        eo[...] = jnp.zeros_like(eo)

    xf = xcur[...]
    h1 = jnp.maximum(_mm_t(xf, W1_ref[0, 0]) + b1_ref[0, 0], 0.0)
    o = _mm_t(h1, W2_ref[0, 0]) + b2_ref[0, 0]
    ii = lax.broadcasted_iota(jnp.int32, (B, E), 1)
    gate = jnp.sum(jnp.where(ii == e, wcomb[...], 0.0), axis=-1, keepdims=True)
    eo[...] = eo[...] + gate * o

    @pl.when(e == E - 1)
    def _fin():
        xn = xf + eo[...]
        xcur[...] = xn

        @pl.when(l == L - 1)
        def _out():
            out_ref[...] = xn


@functools.partial(jax.jit, static_argnames=("interpret",))
def _run(x2, Wp, bp3, W_ih, W_hh, b_ih2, b_hh2, Wr, br3, W1e, b1e4, W2e, b2e4,
         interpret=False):
    grid = (L, E)
    specs = [
        pl.BlockSpec((B, D), lambda l, e: (0, 0)),
        pl.BlockSpec((1, H, D), lambda l, e: (l, 0, 0)),
        pl.BlockSpec((1, 1, H), lambda l, e: (l, 0, 0)),
        pl.BlockSpec((3 * H, H), lambda l, e: (0, 0)),
        pl.BlockSpec((3 * H, H), lambda l, e: (0, 0)),
        pl.BlockSpec((1, 3 * H), lambda l, e: (0, 0)),
        pl.BlockSpec((1, 3 * H), lambda l, e: (0, 0)),
        pl.BlockSpec((1, E, H), lambda l, e: (l, 0, 0)),
        pl.BlockSpec((1, 1, E), lambda l, e: (l, 0, 0)),
        pl.BlockSpec((1, 1, F, D), lambda l, e: (l, e, 0, 0)),
        pl.BlockSpec((1, 1, 1, F), lambda l, e: (l, e, 0, 0)),
        pl.BlockSpec((1, 1, D, F), lambda l, e: (l, e, 0, 0)),
        pl.BlockSpec((1, 1, 1, D), lambda l, e: (l, e, 0, 0)),
    ]
    out = pl.pallas_call(
        _dense_body,
        grid=grid,
        in_specs=specs,
        out_specs=pl.BlockSpec((B, D), lambda l, e: (0, 0)),
        out_shape=jax.ShapeDtypeStruct((B, D), jnp.float32),
        scratch_shapes=[
            pltpu.VMEM((B, D), jnp.float32),
            pltpu.VMEM((B, H), jnp.float32),
            pltpu.VMEM((B, E), jnp.float32),
            pltpu.VMEM((B, D), jnp.float32),
        ],
        compiler_params=pltpu.CompilerParams(
            dimension_semantics=("arbitrary", "arbitrary"),
        ),
        interpret=interpret,
    )(x2, Wp, bp3, W_ih, W_hh, b_ih2, b_hh2, Wr, br3, W1e, b1e4, W2e, b2e4)
    return out


def kernel(x, Wp, bp, W_ih, W_hh, b_ih, b_hh, Wr, br, W1e, b1e, W2e, b2e,
           interpret=False):
    batch, seq, d = x.shape
    x2 = x.reshape(batch * seq, d)
    out = _run(x2, Wp, bp.reshape(L, 1, H), W_ih, W_hh,
               b_ih.reshape(1, 3 * H), b_hh.reshape(1, 3 * H),
               Wr, br.reshape(L, 1, E), W1e, b1e.reshape(L, E, 1, F),
               W2e, b2e.reshape(L, E, 1, D), interpret=interpret)
    return out.reshape(batch, seq, d)


# speedup vs baseline: 1.7088x; 1.7088x over previous
import functools

import jax
import jax.numpy as jnp
from jax import lax
from jax.experimental import pallas as pl
from jax.experimental.pallas import tpu as pltpu

B, D, H, E, L, K = 512, 768, 768, 8, 2, 2
F = 4 * D
NF = 2
FB = F // NF

_DN_T = (((1,), (1,)), ((), ()))


def _mm_t(a, w):
    return lax.dot_general(a, w, _DN_T, preferred_element_type=jnp.float32)


def _dense_body(x_ref, Wp_ref, bp_ref, Wih_ref, Whh_ref, bih_ref, bhh_ref,
                Wr_ref, br_ref, W1_ref, b1_ref, W2_ref, b2_ref,
                out_ref, xcur, h, wcomb, eo):
    l = pl.program_id(0)
    e = pl.program_id(1)
    f = pl.program_id(2)

    @pl.when(jnp.logical_and(l == 0, jnp.logical_and(e == 0, f == 0)))
    def _init():
        xcur[...] = x_ref[...]
        h[...] = jnp.zeros_like(h)

    @pl.when(jnp.logical_and(e == 0, f == 0))
    def _router():
        xf = xcur[...]
        xp = _mm_t(xf, Wp_ref[0]) + bp_ref[0]
        gi = _mm_t(xp, Wih_ref[...]) + bih_ref[0]
        gh = _mm_t(h[...], Whh_ref[...]) + bhh_ref[0]
        i_r, i_z, i_n = gi[:, :H], gi[:, H:2 * H], gi[:, 2 * H:]
        h_r, h_z, h_n = gh[:, :H], gh[:, H:2 * H], gh[:, 2 * H:]
        r = jax.nn.sigmoid(i_r + h_r)
        z = jax.nn.sigmoid(i_z + h_z)
        n = jnp.tanh(i_n + r * h_n)
        hn = (1.0 - z) * n + z * h[...]
        h[...] = hn

        logits = _mm_t(hn, Wr_ref[0]) + br_ref[0]
        m = jnp.max(logits, axis=-1, keepdims=True)
        p = jnp.exp(logits - m)
        p = p / jnp.sum(p, axis=-1, keepdims=True)
        ii = lax.broadcasted_iota(jnp.int32, (B, E), 1)
        m1 = jnp.max(p, axis=-1, keepdims=True)
        i1 = jnp.min(jnp.where(p >= m1, ii, E), axis=-1, keepdims=True)
        p2 = jnp.where(ii == i1, -1e30, p)
        m2 = jnp.max(p2, axis=-1, keepdims=True)
        i2 = jnp.min(jnp.where(p2 >= m2, ii, E), axis=-1, keepdims=True)
        s = m1 + m2
        wcomb[...] = (jnp.where(ii == i1, m1 / s, 0.0)
                      + jnp.where(ii == i2, m2 / s, 0.0))
        eo[...] = jnp.zeros_like(eo)

    xf = xcur[...]
    h1 = jnp.maximum(_mm_t(xf, W1_ref[0, 0]) + b1_ref[0, 0], 0.0)
    o = _mm_t(h1, W2_ref[0, 0])
    ii = lax.broadcasted_iota(jnp.int32, (B, E), 1)
    gate = jnp.sum(jnp.where(ii == e, wcomb[...], 0.0), axis=-1, keepdims=True)
    o = jnp.where(f == 0, o + b2_ref[0, 0], o)
    eo[...] = eo[...] + gate * o

    @pl.when(jnp.logical_and(e == E - 1, f == NF - 1))
    def _fin():
        xn = xf + eo[...]
        xcur[...] = xn

        @pl.when(l == L - 1)
        def _out():
            out_ref[...] = xn


@functools.partial(jax.jit, static_argnames=("interpret",))
def _run(x2, Wp, bp3, W_ih, W_hh, b_ih2, b_hh2, Wr, br3, W1e, b1e4, W2e, b2e4,
         interpret=False):
    grid = (L, E, NF)
    specs = [
        pl.BlockSpec((B, D), lambda l, e, f: (0, 0)),
        pl.BlockSpec((1, H, D), lambda l, e, f: (l, 0, 0)),
        pl.BlockSpec((1, 1, H), lambda l, e, f: (l, 0, 0)),
        pl.BlockSpec((3 * H, H), lambda l, e, f: (0, 0)),
        pl.BlockSpec((3 * H, H), lambda l, e, f: (0, 0)),
        pl.BlockSpec((1, 3 * H), lambda l, e, f: (0, 0)),
        pl.BlockSpec((1, 3 * H), lambda l, e, f: (0, 0)),
        pl.BlockSpec((1, E, H), lambda l, e, f: (l, 0, 0)),
        pl.BlockSpec((1, 1, E), lambda l, e, f: (l, 0, 0)),
        pl.BlockSpec((1, 1, FB, D), lambda l, e, f: (l, e, f, 0)),
        pl.BlockSpec((1, 1, 1, FB), lambda l, e, f: (l, e, 0, f)),
        pl.BlockSpec((1, 1, D, FB), lambda l, e, f: (l, e, 0, f)),
        pl.BlockSpec((1, 1, 1, D), lambda l, e, f: (l, e, 0, 0)),
    ]
    out = pl.pallas_call(
        _dense_body,
        grid=grid,
        in_specs=specs,
        out_specs=pl.BlockSpec((B, D), lambda l, e, f: (0, 0)),
        out_shape=jax.ShapeDtypeStruct((B, D), jnp.float32),
        scratch_shapes=[
            pltpu.VMEM((B, D), jnp.float32),
            pltpu.VMEM((B, H), jnp.float32),
            pltpu.VMEM((B, E), jnp.float32),
            pltpu.VMEM((B, D), jnp.float32),
        ],
        compiler_params=pltpu.CompilerParams(
            dimension_semantics=("arbitrary", "arbitrary", "arbitrary"),
        ),
        interpret=interpret,
    )(x2, Wp, bp3, W_ih, W_hh, b_ih2, b_hh2, Wr, br3, W1e, b1e4, W2e, b2e4)
    return out


def kernel(x, Wp, bp, W_ih, W_hh, b_ih, b_hh, Wr, br, W1e, b1e, W2e, b2e,
           interpret=False):
    batch, seq, d = x.shape
    x2 = x.reshape(batch * seq, d)
    out = _run(x2, Wp, bp.reshape(L, 1, H), W_ih, W_hh,
               b_ih.reshape(1, 3 * H), b_hh.reshape(1, 3 * H),
               Wr, br.reshape(L, 1, E), W1e, b1e.reshape(L, E, 1, F),
               W2e, b2e.reshape(L, E, 1, D), interpret=interpret)
    return out.reshape(batch, seq, d)
